# baseline (speedup 1.0000x reference)
import dataclasses
import functools

import jax
import jax.numpy as jnp
from jax.experimental import pallas as pl
from jax.experimental.pallas import tpu as pltpu
from jax.experimental.pallas import tpu_sc as plsc

_D_MODEL = 2048
_D_SAE = 16384
_K = 64
_N_TOK = 8192

_BM_E = 1024
_BN_E = 1024
_BM_S = 256
_BM_D = 1024
_BK_D = 1024

_R_SC = 2048
_N_WORKERS = 32


def _sortable(pre):
    bits = jax.lax.bitcast_convert_type(pre, jnp.int32)
    return jnp.where(bits >= 0, bits, bits ^ jnp.int32(0x7FFFFFFF))


def _encode_body(x_ref, w_ref, benc_ref, bdec_ref, pre_ref, mpart_ref):
    xc = x_ref[...] - bdec_ref[...]
    pre = (
        jnp.dot(xc, w_ref[...], preferred_element_type=jnp.float32)
        + benc_ref[...]
    )
    pre_ref[...] = pre
    bmax = jnp.max(pre, axis=1, keepdims=True)
    j = pl.program_id(1)

    @pl.when(j == 0)
    def _():
        mpart_ref[...] = bmax

    @pl.when(j != 0)
    def _():
        mpart_ref[...] = jnp.maximum(mpart_ref[...], bmax)


def _select_body(pre_ref, mpart_ref, t_ref, p_ref, sat_ref):
    imin = jnp.iinfo(jnp.int32).min
    s = _sortable(pre_ref[...])
    rows = s.shape[0]
    kf = jnp.float32(_K)

    s_m = _sortable(mpart_ref[...])
    c1 = s_m & jnp.int32(-0x00800000)
    e1 = s_m >> 23
    c2 = jnp.where(e1 > jnp.int32(-256), (e1 - 1) << 23, jnp.int32(imin))
    cnt1 = jnp.sum((s >= c1).astype(jnp.float32), axis=1, keepdims=True)
    cnt2 = jnp.sum((s >= c2).astype(jnp.float32), axis=1, keepdims=True)
    ok = jnp.logical_or(cnt1 >= kf, cnt2 >= kf)
    allok = jnp.sum(ok.astype(jnp.float32)) >= jnp.float32(rows)
    use1 = cnt1 >= kf
    p_init = jnp.where(allok,
                       jnp.where(use1, c1, c2),
                       jnp.full((rows, 1), imin, jnp.int32))
    cnt_init = jnp.where(use1, cnt1, cnt2)
    sat_init = jnp.where(allok, (cnt_init == kf).astype(jnp.float32), 0.0)
    b0 = jnp.where(allok, jnp.int32(22), jnp.int32(31))
    p_ref[...] = p_init
    sat_ref[...] = sat_init

    def cond(carry):
        b, alldone = carry
        return jnp.logical_and(b >= 0, alldone == 0)

    def body(carry):
        b, _ = carry
        p = p_ref[...]
        cand = p + (jnp.int32(1) << b)
        cnt = jnp.sum((s >= cand).astype(jnp.float32), axis=1, keepdims=True)
        keep = cnt >= kf
        p_ref[...] = jnp.where(keep, cand, p)
        sat = jnp.maximum(sat_ref[...], (cnt == kf).astype(jnp.float32))
        sat_ref[...] = sat
        alldone = (jnp.sum(sat) >= jnp.float32(rows)).astype(jnp.int32)
        return b - 1, alldone

    jax.lax.while_loop(cond, body, (b0, jnp.int32(0)))
    t_ref[...] = p_ref[...]


def _sc_select_body(pre_hbm, t_hbm, frow, srow, stage, sem):
    imin = jnp.int32(-2147483648)
    ki = jnp.int32(_K)
    nchunk = _D_SAE // 16
    rows_per = _R_SC // _N_WORKERS
    wid = jax.lax.axis_index("s") * 2 + jax.lax.axis_index("c")
    base = (_N_TOK - _R_SC) + wid * rows_per

    @pl.loop(0, rows_per // 16)
    def _group(g):
        def row_body(r16, tvec):
            row = base + g * 16 + r16
            pltpu.async_copy(pre_hbm.at[row], frow, sem).wait()

            def s_body(j, m):
                f = frow[pl.ds(j * 16, 16)]
                bits = jax.lax.bitcast_convert_type(f, jnp.int32)
                s = jnp.where(bits >= 0, bits, bits ^ jnp.int32(0x7FFFFFFF))
                srow[pl.ds(j * 16, 16)] = s
                return jnp.maximum(m, s)

            mv = jax.lax.fori_loop(
                0, nchunk, s_body, jnp.full((16,), imin, jnp.int32))
            sm = jnp.max(mv)
            c1 = sm & jnp.int32(-0x00800000)
            e1 = sm >> 23
            c2 = jnp.where(e1 > jnp.int32(-256), (e1 - 1) << 23, imin)

            def probe_body(j, acc):
                a1, a2 = acc
                s = srow[pl.ds(j * 16, 16)]
                return (a1 + (s >= c1).astype(jnp.int32),
                        a2 + (s >= c2).astype(jnp.int32))

            z16 = jnp.zeros((16,), jnp.int32)
            a1, a2 = jax.lax.fori_loop(0, nchunk, probe_body, (z16, z16))
            cnt1 = jnp.sum(a1)
            cnt2 = jnp.sum(a2)
            use1 = cnt1 >= ki
            ok = jnp.logical_or(use1, cnt2 >= ki)
            p0 = jnp.where(use1, c1, jnp.where(cnt2 >= ki, c2, imin))
            b0 = jnp.where(ok, jnp.int32(22), jnp.int32(31))
            cnt0 = jnp.where(use1, cnt1, cnt2)
            sat0 = jnp.logical_and(ok, cnt0 == ki)

            def wcond(carry):
                b, _, sat = carry
                return jnp.logical_and(b >= 0, jnp.logical_not(sat))

            def wbody(carry):
                b, p, _ = carry
                cand = p + (jnp.int32(1) << b)

                def cnt_body(j, a):
                    s = srow[pl.ds(j * 16, 16)]
                    return a + (s >= cand).astype(jnp.int32)

                cnt = jnp.sum(jax.lax.fori_loop(0, nchunk, cnt_body, z16))
                keep = cnt >= ki
                return b - 1, jnp.where(keep, cand, p), cnt == ki

            _, pf, _ = jax.lax.while_loop(wcond, wbody, (b0, p0, sat0))
            lane = jax.lax.iota(jnp.int32, 16)
            return jnp.where(lane == r16, pf, tvec)

        tvec = jax.lax.fori_loop(0, 16, row_body, jnp.zeros((16,), jnp.int32))
        stage[...] = tvec
        pltpu.sync_copy(
            stage, t_hbm.at[pl.ds(wid * rows_per + g * 16, 16)])


def _decode_body(pre_ref, t_ref, w_ref, bdec_ref, out_ref):
    k = pl.program_id(1)
    pre = pre_ref[...]
    s = _sortable(pre)
    acts = jnp.where(s >= t_ref[...], jnp.maximum(pre, 0.0), 0.0)
    contrib = jnp.dot(acts, w_ref[...], preferred_element_type=jnp.float32)

    @pl.when(k == 0)
    def _():
        out_ref[...] = contrib + bdec_ref[...]

    @pl.when(k != 0)
    def _():
        out_ref[...] += contrib


def kernel(x, W_enc, W_dec, b_enc, b_dec):
    n_tok, d_model = x.shape
    d_sae = W_enc.shape[1]
    benc2 = b_enc.reshape(1, d_sae)
    bdec2 = b_dec.reshape(1, d_model)

    n_jb = d_sae // _BN_E
    pre, mpart = pl.pallas_call(
        _encode_body,
        grid=(n_tok // _BM_E, n_jb),
        in_specs=[
            pl.BlockSpec((_BM_E, d_model), lambda i, j: (i, 0)),
            pl.BlockSpec((d_model, _BN_E), lambda i, j: (0, j)),
            pl.BlockSpec((1, _BN_E), lambda i, j: (0, j)),
            pl.BlockSpec((1, d_model), lambda i, j: (0, 0)),
        ],
        out_specs=[
            pl.BlockSpec((_BM_E, _BN_E), lambda i, j: (i, j)),
            pl.BlockSpec((_BM_E, 1), lambda i, j: (i, 0)),
        ],
        out_shape=[
            jax.ShapeDtypeStruct((n_tok, d_sae), jnp.float32),
            jax.ShapeDtypeStruct((n_tok, 1), jnp.float32),
        ],
        compiler_params=pltpu.CompilerParams(
            dimension_semantics=("parallel", "parallel"),
        ),
    )(x, W_enc, benc2, bdec2)

    r_tc = n_tok - _R_SC

    t_tc = pl.pallas_call(
        _select_body,
        grid=(r_tc // _BM_S,),
        in_specs=[
            pl.BlockSpec((_BM_S, d_sae), lambda i: (i, 0)),
            pl.BlockSpec((_BM_S, 1), lambda i: (i, 0)),
        ],
        out_specs=pl.BlockSpec((_BM_S, 1), lambda i: (i, 0)),
        out_shape=jax.ShapeDtypeStruct((r_tc, 1), jnp.int32),
        scratch_shapes=[
            pltpu.VMEM((_BM_S, 1), jnp.int32),
            pltpu.VMEM((_BM_S, 1), jnp.float32),
        ],
        compiler_params=pltpu.CompilerParams(
            dimension_semantics=("parallel",),
        ),
    )(pre, mpart)

    sc_cp = pltpu.CompilerParams()
    if "needs_layout_passes" in pltpu.CompilerParams.__dataclass_fields__:
        sc_cp = dataclasses.replace(sc_cp, needs_layout_passes=False)
    t_sc = pl.kernel(
        _sc_select_body,
        out_type=jax.ShapeDtypeStruct((_R_SC,), jnp.int32),
        mesh=plsc.VectorSubcoreMesh(core_axis_name="c", subcore_axis_name="s"),
        compiler_params=sc_cp,
        scratch_types=[
            pltpu.VMEM((d_sae,), jnp.float32),
            pltpu.VMEM((d_sae,), jnp.int32),
            pltpu.VMEM((16,), jnp.int32),
            pltpu.SemaphoreType.DMA,
        ],
    )(pre)

    def _decode_call(t_arg, n_rows, i_off):
        return pl.pallas_call(
            _decode_body,
            grid=(n_rows // _BM_D, d_sae // _BK_D),
            in_specs=[
                pl.BlockSpec((_BM_D, _BK_D), lambda i, k: (i + i_off, k)),
                pl.BlockSpec((_BM_D, 1), lambda i, k: (i, 0)),
                pl.BlockSpec((_BK_D, d_model), lambda i, k: (k, 0)),
                pl.BlockSpec((1, d_model), lambda i, k: (0, 0)),
            ],
            out_specs=pl.BlockSpec((_BM_D, d_model), lambda i, k: (i, 0)),
            out_shape=jax.ShapeDtypeStruct((n_rows, d_model), jnp.float32),
            compiler_params=pltpu.CompilerParams(
                dimension_semantics=("parallel", "arbitrary"),
            ),
        )(pre, t_arg, W_dec, bdec2)

    out_tc = _decode_call(t_tc, r_tc, 0)
    out_sc = _decode_call(t_sc.reshape(_R_SC, 1), _R_SC, r_tc // _BM_D)
    return jnp.concatenate([out_tc, out_sc], axis=0)


# speedup vs baseline: 1.4827x; 1.4827x over previous
import dataclasses
import functools

import jax
import jax.numpy as jnp
from jax.experimental import pallas as pl
from jax.experimental.pallas import tpu as pltpu
from jax.experimental.pallas import tpu_sc as plsc

_D_MODEL = 2048
_D_SAE = 16384
_K = 64
_N_TOK = 8192

_BM_E = 1024
_BN_E = 1024
_BM_S = 256
_BM_D = 1024
_BK_D = 1024

_R_SC = 2048
_N_WORKERS = 32
_SC_UNROLL = 8


def _sortable(pre):
    bits = jax.lax.bitcast_convert_type(pre, jnp.int32)
    return jnp.where(bits >= 0, bits, bits ^ jnp.int32(0x7FFFFFFF))


def _encode_body(x_ref, w_ref, benc_ref, bdec_ref, pre_ref, mpart_ref):
    xc = x_ref[...] - bdec_ref[...]
    pre = (
        jnp.dot(xc, w_ref[...], preferred_element_type=jnp.float32)
        + benc_ref[...]
    )
    pre_ref[...] = pre
    bmax = jnp.max(pre, axis=1, keepdims=True)
    j = pl.program_id(1)

    @pl.when(j == 0)
    def _():
        mpart_ref[...] = bmax

    @pl.when(j != 0)
    def _():
        mpart_ref[...] = jnp.maximum(mpart_ref[...], bmax)


def _select_body(pre_ref, mpart_ref, t_ref, p_ref, sat_ref):
    imin = jnp.iinfo(jnp.int32).min
    s = _sortable(pre_ref[...])
    rows = s.shape[0]
    kf = jnp.float32(_K)

    s_m = _sortable(mpart_ref[...])
    c1 = s_m & jnp.int32(-0x00800000)
    e1 = s_m >> 23
    c2 = jnp.where(e1 > jnp.int32(-256), (e1 - 1) << 23, jnp.int32(imin))
    cnt1 = jnp.sum((s >= c1).astype(jnp.float32), axis=1, keepdims=True)
    cnt2 = jnp.sum((s >= c2).astype(jnp.float32), axis=1, keepdims=True)
    ok = jnp.logical_or(cnt1 >= kf, cnt2 >= kf)
    allok = jnp.sum(ok.astype(jnp.float32)) >= jnp.float32(rows)
    use1 = cnt1 >= kf
    p_init = jnp.where(allok,
                       jnp.where(use1, c1, c2),
                       jnp.full((rows, 1), imin, jnp.int32))
    cnt_init = jnp.where(use1, cnt1, cnt2)
    sat_init = jnp.where(allok, (cnt_init == kf).astype(jnp.float32), 0.0)
    b0 = jnp.where(allok, jnp.int32(22), jnp.int32(31))
    p_ref[...] = p_init
    sat_ref[...] = sat_init

    def cond(carry):
        b, alldone = carry
        return jnp.logical_and(b >= 0, alldone == 0)

    def body(carry):
        b, _ = carry
        p = p_ref[...]
        cand = p + (jnp.int32(1) << b)
        cnt = jnp.sum((s >= cand).astype(jnp.float32), axis=1, keepdims=True)
        keep = cnt >= kf
        p_ref[...] = jnp.where(keep, cand, p)
        sat = jnp.maximum(sat_ref[...], (cnt == kf).astype(jnp.float32))
        sat_ref[...] = sat
        alldone = (jnp.sum(sat) >= jnp.float32(rows)).astype(jnp.int32)
        return b - 1, alldone

    jax.lax.while_loop(cond, body, (b0, jnp.int32(0)))
    t_ref[...] = p_ref[...]


def _sc_select_body(pre_hbm, t_hbm, frow, srow, stage, sem):
    imin = jnp.int32(-2147483648)
    ki = jnp.int32(_K)
    nchunk = _D_SAE // 16
    rows_per = _R_SC // _N_WORKERS
    wid = jax.lax.axis_index("s") * 2 + jax.lax.axis_index("c")
    base = (_N_TOK - _R_SC) + wid * rows_per

    @pl.loop(0, rows_per // 16)
    def _group(g):
        def row_body(r16, tvec):
            row = base + g * 16 + r16
            pltpu.async_copy(pre_hbm.at[row], frow, sem).wait()

            def s_body(j, m):
                for u in range(_SC_UNROLL):
                    off = (j * _SC_UNROLL + u) * 16
                    f = frow[pl.ds(off, 16)]
                    bits = jax.lax.bitcast_convert_type(f, jnp.int32)
                    s = jnp.where(bits >= 0, bits,
                                  bits ^ jnp.int32(0x7FFFFFFF))
                    srow[pl.ds(off, 16)] = s
                    m = jnp.maximum(m, s)
                return m

            mv = jax.lax.fori_loop(
                0, nchunk // _SC_UNROLL, s_body,
                jnp.full((16,), imin, jnp.int32))
            sm = jnp.max(mv)
            c1 = sm & jnp.int32(-0x00800000)
            e1 = sm >> 23
            c2 = jnp.where(e1 > jnp.int32(-256), (e1 - 1) << 23, imin)

            def probe_body(j, acc):
                a1, a2 = acc
                for u in range(_SC_UNROLL):
                    s = srow[pl.ds((j * _SC_UNROLL + u) * 16, 16)]
                    a1 = a1 + (s >= c1).astype(jnp.int32)
                    a2 = a2 + (s >= c2).astype(jnp.int32)
                return a1, a2

            z16 = jnp.zeros((16,), jnp.int32)
            a1, a2 = jax.lax.fori_loop(
                0, nchunk // _SC_UNROLL, probe_body, (z16, z16))
            cnt1 = jnp.sum(a1)
            cnt2 = jnp.sum(a2)
            use1 = cnt1 >= ki
            ok = jnp.logical_or(use1, cnt2 >= ki)
            p0 = jnp.where(use1, c1, jnp.where(cnt2 >= ki, c2, imin))
            b0 = jnp.where(ok, jnp.int32(22), jnp.int32(31))
            cnt0 = jnp.where(use1, cnt1, cnt2)
            sat0 = jnp.logical_and(ok, cnt0 == ki)

            def wcond(carry):
                b, _, sat = carry
                return jnp.logical_and(b >= 0, jnp.logical_not(sat))

            def wbody(carry):
                b, p, _ = carry
                cand = p + (jnp.int32(1) << b)

                def cnt_body(j, a):
                    for u in range(_SC_UNROLL):
                        s = srow[pl.ds((j * _SC_UNROLL + u) * 16, 16)]
                        a = a + (s >= cand).astype(jnp.int32)
                    return a

                cnt = jnp.sum(jax.lax.fori_loop(
                    0, nchunk // _SC_UNROLL, cnt_body, z16))
                keep = cnt >= ki
                return b - 1, jnp.where(keep, cand, p), cnt == ki

            _, pf, _ = jax.lax.while_loop(wcond, wbody, (b0, p0, sat0))
            lane = jax.lax.iota(jnp.int32, 16)
            return jnp.where(lane == r16, pf, tvec)

        tvec = jax.lax.fori_loop(0, 16, row_body, jnp.zeros((16,), jnp.int32))
        stage[...] = tvec
        pltpu.sync_copy(
            stage, t_hbm.at[pl.ds(wid * rows_per + g * 16, 16)])


def _decode_body(pre_ref, t_ref, w_ref, bdec_ref, out_ref):
    k = pl.program_id(1)
    pre = pre_ref[...]
    s = _sortable(pre)
    acts = jnp.where(s >= t_ref[...], jnp.maximum(pre, 0.0), 0.0)
    contrib = jnp.dot(acts, w_ref[...], preferred_element_type=jnp.float32)

    @pl.when(k == 0)
    def _():
        out_ref[...] = contrib + bdec_ref[...]

    @pl.when(k != 0)
    def _():
        out_ref[...] += contrib


def kernel(x, W_enc, W_dec, b_enc, b_dec):
    n_tok, d_model = x.shape
    d_sae = W_enc.shape[1]
    benc2 = b_enc.reshape(1, d_sae)
    bdec2 = b_dec.reshape(1, d_model)

    n_jb = d_sae // _BN_E
    pre, mpart = pl.pallas_call(
        _encode_body,
        grid=(n_tok // _BM_E, n_jb),
        in_specs=[
            pl.BlockSpec((_BM_E, d_model), lambda i, j: (i, 0)),
            pl.BlockSpec((d_model, _BN_E), lambda i, j: (0, j)),
            pl.BlockSpec((1, _BN_E), lambda i, j: (0, j)),
            pl.BlockSpec((1, d_model), lambda i, j: (0, 0)),
        ],
        out_specs=[
            pl.BlockSpec((_BM_E, _BN_E), lambda i, j: (i, j)),
            pl.BlockSpec((_BM_E, 1), lambda i, j: (i, 0)),
        ],
        out_shape=[
            jax.ShapeDtypeStruct((n_tok, d_sae), jnp.float32),
            jax.ShapeDtypeStruct((n_tok, 1), jnp.float32),
        ],
        compiler_params=pltpu.CompilerParams(
            dimension_semantics=("parallel", "parallel"),
        ),
    )(x, W_enc, benc2, bdec2)

    r_tc = n_tok - _R_SC

    t_tc = pl.pallas_call(
        _select_body,
        grid=(r_tc // _BM_S,),
        in_specs=[
            pl.BlockSpec((_BM_S, d_sae), lambda i: (i, 0)),
            pl.BlockSpec((_BM_S, 1), lambda i: (i, 0)),
        ],
        out_specs=pl.BlockSpec((_BM_S, 1), lambda i: (i, 0)),
        out_shape=jax.ShapeDtypeStruct((r_tc, 1), jnp.int32),
        scratch_shapes=[
            pltpu.VMEM((_BM_S, 1), jnp.int32),
            pltpu.VMEM((_BM_S, 1), jnp.float32),
        ],
        compiler_params=pltpu.CompilerParams(
            dimension_semantics=("parallel",),
        ),
    )(pre, mpart)

    sc_cp = pltpu.CompilerParams()
    if "needs_layout_passes" in pltpu.CompilerParams.__dataclass_fields__:
        sc_cp = dataclasses.replace(sc_cp, needs_layout_passes=False)
    t_sc = pl.kernel(
        _sc_select_body,
        out_type=jax.ShapeDtypeStruct((_R_SC,), jnp.int32),
        mesh=plsc.VectorSubcoreMesh(core_axis_name="c", subcore_axis_name="s"),
        compiler_params=sc_cp,
        scratch_types=[
            pltpu.VMEM((d_sae,), jnp.float32),
            pltpu.VMEM((d_sae,), jnp.int32),
            pltpu.VMEM((16,), jnp.int32),
            pltpu.SemaphoreType.DMA,
        ],
    )(pre)

    def _decode_call(t_arg, n_rows, i_off):
        return pl.pallas_call(
            _decode_body,
            grid=(n_rows // _BM_D, d_sae // _BK_D),
            in_specs=[
                pl.BlockSpec((_BM_D, _BK_D), lambda i, k: (i + i_off, k)),
                pl.BlockSpec((_BM_D, 1), lambda i, k: (i, 0)),
                pl.BlockSpec((_BK_D, d_model), lambda i, k: (k, 0)),
                pl.BlockSpec((1, d_model), lambda i, k: (0, 0)),
            ],
            out_specs=pl.BlockSpec((_BM_D, d_model), lambda i, k: (i, 0)),
            out_shape=jax.ShapeDtypeStruct((n_rows, d_model), jnp.float32),
            compiler_params=pltpu.CompilerParams(
                dimension_semantics=("parallel", "arbitrary"),
            ),
        )(pre, t_arg, W_dec, bdec2)

    out_tc = _decode_call(t_tc, r_tc, 0)
    out_sc = _decode_call(t_sc.reshape(_R_SC, 1), _R_SC, r_tc // _BM_D)
    return jnp.concatenate([out_tc, out_sc], axis=0)
